# Initial kernel scaffold; baseline (speedup 1.0000x reference)
import jax
import jax.numpy as jnp
from jax.experimental import pallas as pl


def kernel(data, edge_index, edge_weight, var_u, var_c, var_w, var_b):
    raise NotImplementedError("write your pallas kernel here")


# speedup vs baseline: 1.1082x; 1.1082x over previous
import functools

import jax
import jax.numpy as jnp
from jax import lax
from jax.experimental import pallas as pl
from jax.experimental.pallas import tpu as pltpu
from jax.experimental.pallas import tpu_sc as plsc

NC = 2
NS = 16
LN = 16
NW = NC * NS


def _splat(vec, j):
    idx = jnp.full((LN,), j, dtype=jnp.int32)
    return lax.gather(
        vec, idx[:, None],
        dimension_numbers=lax.GatherDimensionNumbers(
            offset_dims=(), collapsed_slice_dims=(0,), start_index_map=(0,)),
        slice_sizes=(1,), mode=lax.GatherScatterMode.PROMISE_IN_BOUNDS)


def _xu_tc(data, var_u):
    V, C = data.shape
    M = var_u.shape[1]

    def body(d_ref, u_ref, o_ref):
        o_ref[...] = jnp.dot(d_ref[...], u_ref[...],
                             preferred_element_type=jnp.float32)

    return pl.pallas_call(
        body,
        out_shape=jax.ShapeDtypeStruct((V, M), jnp.float32),
    )(data, var_u)


def _coef_sc(xu_flat, dst, src, ew, c_pad, M, E, V):
    EPT = E // NW
    CHUNK = 2000
    NCH = EPT // CHUNK
    mesh = plsc.VectorSubcoreMesh(core_axis_name="c", subcore_axis_name="s")

    @functools.partial(
        pl.kernel,
        out_type=jax.ShapeDtypeStruct((M * E,), jnp.float32),
        mesh=mesh,
        compiler_params=pltpu.CompilerParams(needs_layout_passes=False),
        scratch_types=[
            pltpu.VMEM((V * M,), jnp.float32),
            pltpu.VMEM((LN,), jnp.float32),
            pltpu.VMEM((CHUNK,), jnp.int32),
            pltpu.VMEM((CHUNK,), jnp.int32),
            pltpu.VMEM((CHUNK,), jnp.float32),
            pltpu.VMEM((M * CHUNK,), jnp.float32),
        ],
    )
    def k(xu_hbm, dst_hbm, src_hbm, ew_hbm, c_hbm, coef_hbm,
          xu_v, c_v, dst_v, src_v, ew_v, co_v):
        wid = lax.axis_index("s") * NC + lax.axis_index("c")
        ebase = wid * EPT
        pltpu.sync_copy(xu_hbm, xu_v)
        pltpu.sync_copy(c_hbm, c_v)
        cvec = c_v[...]
        csp = [_splat(cvec, m) for m in range(M)]

        def chunk_body(kk, _):
            base = ebase + kk * CHUNK
            pltpu.sync_copy(dst_hbm.at[pl.ds(base, CHUNK)], dst_v)
            pltpu.sync_copy(src_hbm.at[pl.ds(base, CHUNK)], src_v)
            pltpu.sync_copy(ew_hbm.at[pl.ds(base, CHUNK)], ew_v)

            def vec_body(i, _):
                sl = pl.ds(i * LN, LN)
                d16 = dst_v[sl] * M
                s16 = src_v[sl] * M
                lo = []
                for m in range(M):
                    lu = plsc.load_gather(xu_v, [d16 + m])
                    lv = plsc.load_gather(xu_v, [s16 + m])
                    lo.append(lu - lv + csp[m])
                mx = functools.reduce(jnp.maximum, lo)
                es = [jnp.exp(l - mx) for l in lo]
                tot = functools.reduce(lambda a, b: a + b, es)
                r = ew_v[sl] / tot
                for m in range(M):
                    co_v[pl.ds(m * CHUNK + i * LN, LN)] = es[m] * r
                return 0

            lax.fori_loop(0, CHUNK // LN, vec_body, 0)
            for m in range(M):
                pltpu.sync_copy(co_v.at[pl.ds(m * CHUNK, CHUNK)],
                                coef_hbm.at[pl.ds(m * E + base, CHUNK)])
            return 0

        lax.fori_loop(0, NCH, chunk_body, 0)

    return k(xu_flat, dst, src, ew, c_pad)


def _agg_sc(dst, src, coef, data, M, E, V, VP):
    EPS = E // NS
    BB = 80
    CH = 2000
    NBC = CH // BB
    NCH = EPS // CH
    RPT = VP // NS
    RNDS = M // NC
    C = data.shape[1]
    KV = C // LN
    mesh = plsc.VectorSubcoreMesh(core_axis_name="c", subcore_axis_name="s")

    @functools.partial(
        pl.kernel,
        out_type=jax.ShapeDtypeStruct((M, VP, C), jnp.float32),
        mesh=mesh,
        compiler_params=pltpu.CompilerParams(needs_layout_passes=False),
        scratch_types=[
            pltpu.VMEM((NBC, BB), jnp.int32),
            pltpu.VMEM((NBC, BB), jnp.int32),
            pltpu.VMEM((CH,), jnp.float32),
            pltpu.VMEM((BB, C), jnp.float32),
            pltpu.VMEM((BB, C), jnp.float32),
            pltpu.VMEM_SHARED((VP, C), jnp.float32),
            pltpu.SemaphoreType.DMA,
        ],
    )
    def k(dst_hbm, src_hbm, coef_hbm, data_hbm, agg_hbm,
          si_v, di_v, cf_v, rows_v, con_v, agg_sh, sem):
        cid = lax.axis_index("c")
        sid = lax.axis_index("s")
        ebase = sid * EPS
        zv = jnp.zeros((LN,), jnp.float32)

        for r in range(RNDS):
            m = r * NC + cid

            def zb(g, _):
                for kk in range(KV):
                    con_v[g, pl.ds(kk * LN, LN)] = zv
                return 0
            lax.fori_loop(0, BB, zb, 0)
            for j in range(RPT // BB):
                pltpu.sync_copy(con_v,
                                agg_sh.at[pl.ds(sid * RPT + j * BB, BB)])
            plsc.subcore_barrier()

            def chunk_body(cc, _):
                cbase = ebase + cc * CH

                def ld_idx(b, _):
                    eoff = cbase + b * BB
                    pltpu.sync_copy(src_hbm.at[pl.ds(eoff, BB)], si_v.at[b])
                    pltpu.sync_copy(dst_hbm.at[pl.ds(eoff, BB)], di_v.at[b])
                    return 0
                lax.fori_loop(0, NBC, ld_idx, 0)
                pltpu.sync_copy(coef_hbm.at[pl.ds(m * E + cbase, CH)], cf_v)

                def batch_body(b, _):
                    pltpu.async_copy(
                        data_hbm.at[si_v.at[b]], rows_v, sem).wait()

                    def grp(g, _):
                        cf16 = cf_v[pl.ds(b * BB + g * LN, LN)]
                        for j in range(LN):
                            cs = _splat(cf16, j)
                            e = g * LN + j
                            for kk in range(KV):
                                sl = pl.ds(kk * LN, LN)
                                con_v[e, sl] = rows_v[e, sl] * cs
                        return 0

                    lax.fori_loop(0, BB // LN, grp, 0)
                    pltpu.sync_copy(con_v, agg_sh.at[di_v.at[b]], add=True)
                    return 0

                lax.fori_loop(0, NBC, batch_body, 0)
                return 0

            lax.fori_loop(0, NCH, chunk_body, 0)
            plsc.subcore_barrier()

            for j in range(RPT // BB):
                row = sid * RPT + j * BB
                pltpu.sync_copy(agg_sh.at[pl.ds(row, BB)], rows_v)
                pltpu.sync_copy(rows_v, agg_hbm.at[m, pl.ds(row, BB)])
            plsc.subcore_barrier()

    return k(dst, src, coef, data)


def _out_tc(agg, var_w, var_b, V):
    M, VP, C = agg.shape
    O = var_w.shape[2]
    BV = 256
    NBV = (V + BV - 1) // BV

    def body(a_ref, w_ref, b_ref, o_ref):
        m = pl.program_id(1)
        part = jnp.dot(a_ref[0], w_ref[0], preferred_element_type=jnp.float32)

        @pl.when(m == 0)
        def _():
            o_ref[...] = part + b_ref[...]

        @pl.when(m > 0)
        def _():
            o_ref[...] += part

    return pl.pallas_call(
        body,
        grid=(NBV, M),
        in_specs=[
            pl.BlockSpec((1, BV, C), lambda i, m: (m, i, 0)),
            pl.BlockSpec((1, C, O), lambda i, m: (m, 0, 0)),
            pl.BlockSpec((1, O), lambda i, m: (0, 0)),
        ],
        out_specs=pl.BlockSpec((BV, O), lambda i, m: (i, 0)),
        out_shape=jax.ShapeDtypeStruct((V, O), jnp.float32),
    )(agg, var_w, var_b[None])


def kernel(data, edge_index, edge_weight, var_u, var_c, var_w, var_b):
    V, C = data.shape
    M = var_u.shape[1]
    E = edge_index.shape[1]
    assert E % (NW * 2000) == 0 and V * M * 4 <= 500_000
    dst = edge_index[0].astype(jnp.int32)
    src = edge_index[1].astype(jnp.int32)
    ew = edge_weight.astype(jnp.float32)
    c_pad = jnp.zeros((LN,), jnp.float32).at[:M].set(var_c)

    x_u = _xu_tc(data, var_u)
    coef = _coef_sc(x_u.reshape(V * M), dst, src, ew, c_pad, M, E, V)

    VP = ((V + 255) // 256) * 256
    agg = _agg_sc(dst, src, coef, data, M, E, V, VP)
    return _out_tc(agg, var_w, var_b, V)
